# baseline (speedup 1.0000x reference)
import functools

import jax
import jax.numpy as jnp
from jax import lax
from jax.experimental import pallas as pl
from jax.experimental.pallas import tpu as pltpu
from jax.experimental.pallas import tpu_sc as plsc

N = 10000
E = 320000
D = 128
H = 128
EDIM = 4
L = 4
OUT = 128
B = 16

NBLK = 2000
EBLK = 2560


def _silu(x):
    return x * jax.nn.sigmoid(x)


def _ln(x, g, b):
    mu = jnp.mean(x, axis=-1, keepdims=True)
    var = jnp.mean((x - mu) ** 2, axis=-1, keepdims=True)
    return (x - mu) * jax.lax.rsqrt(var + 1e-5) * g + b


def _input_body(nf, w, b, g, beta, o):
    h = jnp.dot(nf[...], w[...], preferred_element_type=jnp.float32) + b[...]
    o[...] = _silu(_ln(h, g[...], beta[...]))


def _input_proj(nf, W_in, b_in, g, beta):
    grid = (N // NBLK,)
    return pl.pallas_call(
        _input_body,
        grid=grid,
        in_specs=[
            pl.BlockSpec((NBLK, D), lambda i: (i, 0)),
            pl.BlockSpec((D, H), lambda i: (0, 0)),
            pl.BlockSpec((1, H), lambda i: (0, 0)),
            pl.BlockSpec((1, H), lambda i: (0, 0)),
            pl.BlockSpec((1, H), lambda i: (0, 0)),
        ],
        out_specs=pl.BlockSpec((NBLK, H), lambda i: (i, 0)),
        out_shape=jax.ShapeDtypeStruct((N, H), jnp.float32),
    )(nf, W_in, b_in.reshape(1, H), g.reshape(1, H), beta.reshape(1, H))


def _pq_body(x, wa, wb, p, q):
    xv = x[...]
    p[...] = jnp.dot(xv, wa[...], preferred_element_type=jnp.float32)
    q[...] = jnp.dot(xv, wb[...], preferred_element_type=jnp.float32)


def _pq(x, W1a, W1b):
    grid = (N // NBLK,)
    return pl.pallas_call(
        _pq_body,
        grid=grid,
        in_specs=[
            pl.BlockSpec((NBLK, H), lambda i: (i, 0)),
            pl.BlockSpec((H, H), lambda i: (0, 0)),
            pl.BlockSpec((H, H), lambda i: (0, 0)),
        ],
        out_specs=[
            pl.BlockSpec((NBLK, H), lambda i: (i, 0)),
            pl.BlockSpec((NBLK, H), lambda i: (i, 0)),
        ],
        out_shape=[
            jax.ShapeDtypeStruct((N, H), jnp.float32),
            jax.ShapeDtypeStruct((N, H), jnp.float32),
        ],
    )(x, W1a, W1b)


def _edge_body(gp, gq, ea, w1c, b1, w2, b2, w3, b3, m_out):
    a = jnp.dot(ea[...], w1c[...], preferred_element_type=jnp.float32) + b1[...]
    m1 = _silu(gp[...] + gq[...] + a)
    m2 = _silu(jnp.dot(m1, w2[...], preferred_element_type=jnp.float32) + b2[...])
    m3 = jnp.dot(m2, w3[...], preferred_element_type=jnp.float32) + b3[...]
    dist = ea[:, 0:1]
    m_out[...] = m3 / (dist + 1e-8)


def _edge_mlp(gp, gq, ea, W1c, b1, W2, b2, W3, b3):
    ecnt = gp.shape[0]
    grid = (ecnt // EBLK,)
    return pl.pallas_call(
        _edge_body,
        grid=grid,
        in_specs=[
            pl.BlockSpec((EBLK, H), lambda i: (i, 0)),
            pl.BlockSpec((EBLK, H), lambda i: (i, 0)),
            pl.BlockSpec((EBLK, EDIM), lambda i: (i, 0)),
            pl.BlockSpec((EDIM, H), lambda i: (0, 0)),
            pl.BlockSpec((1, H), lambda i: (0, 0)),
            pl.BlockSpec((H, H), lambda i: (0, 0)),
            pl.BlockSpec((1, H), lambda i: (0, 0)),
            pl.BlockSpec((H, H), lambda i: (0, 0)),
            pl.BlockSpec((1, H), lambda i: (0, 0)),
        ],
        out_specs=pl.BlockSpec((EBLK, H), lambda i: (i, 0)),
        out_shape=jax.ShapeDtypeStruct((ecnt, H), jnp.float32),
    )(gp, gq, ea, W1c, b1.reshape(1, H), W2, b2.reshape(1, H), W3,
      b3.reshape(1, H))


def _node_body(x, p0, p1, p2, p3, c0, c1, wa, wb, b1, w2, b2, g, beta, xo):
    xv = x[...]
    cnt = c0[:, 0:1] + c1[:, 0:1]
    deg = jnp.maximum(cnt, 1.0)
    agg = (p0[...] + p1[...] + p2[...] + p3[...]) / deg
    h = _silu(jnp.dot(xv, wa[...], preferred_element_type=jnp.float32)
              + jnp.dot(agg, wb[...], preferred_element_type=jnp.float32)
              + b1[...])
    h2 = jnp.dot(h, w2[...], preferred_element_type=jnp.float32) + b2[...]
    xo[...] = _ln(h2 + xv, g[...], beta[...])


def _node_update(x, p0, p1, p2, p3, c0, c1, nW1a, nW1b, b1, W2, b2, g, beta):
    grid = (N // NBLK,)
    return pl.pallas_call(
        _node_body,
        grid=grid,
        in_specs=[
            pl.BlockSpec((NBLK, H), lambda i: (i, 0)),
            pl.BlockSpec((NBLK, H), lambda i: (i, 0)),
            pl.BlockSpec((NBLK, H), lambda i: (i, 0)),
            pl.BlockSpec((NBLK, H), lambda i: (i, 0)),
            pl.BlockSpec((NBLK, H), lambda i: (i, 0)),
            pl.BlockSpec((NBLK, H), lambda i: (i, 0)),
            pl.BlockSpec((NBLK, H), lambda i: (i, 0)),
            pl.BlockSpec((H, H), lambda i: (0, 0)),
            pl.BlockSpec((H, H), lambda i: (0, 0)),
            pl.BlockSpec((1, H), lambda i: (0, 0)),
            pl.BlockSpec((H, H), lambda i: (0, 0)),
            pl.BlockSpec((1, H), lambda i: (0, 0)),
            pl.BlockSpec((1, H), lambda i: (0, 0)),
            pl.BlockSpec((1, H), lambda i: (0, 0)),
        ],
        out_specs=pl.BlockSpec((NBLK, H), lambda i: (i, 0)),
        out_shape=jax.ShapeDtypeStruct((N, H), jnp.float32),
    )(x, p0, p1, p2, p3, c0, c1, nW1a, nW1b, b1.reshape(1, H), W2,
      b2.reshape(1, H), g.reshape(1, H), beta.reshape(1, H))


def _out_body(x, batch, w1, b1, g, beta, w2, b2, o, cnt):
    i = pl.program_id(0)
    y = _silu(_ln(jnp.dot(x[...], w1[...], preferred_element_type=jnp.float32)
                  + b1[...], g[...], beta[...]))
    z = jnp.dot(y, w2[...], preferred_element_type=jnp.float32) + b2[...]
    bb = batch[...]
    iota = jax.lax.broadcasted_iota(jnp.int32, (NBLK, B), 1)
    oh = (bb == iota).astype(jnp.float32)
    pooled = jax.lax.dot_general(oh, z, (((0,), (0,)), ((), ())),
                                 preferred_element_type=jnp.float32)
    ones = jnp.ones((NBLK, OUT), jnp.float32)
    c = jax.lax.dot_general(oh, ones, (((0,), (0,)), ((), ())),
                            preferred_element_type=jnp.float32)

    @pl.when(i == 0)
    def _():
        o[...] = jnp.zeros_like(o)
        cnt[...] = jnp.zeros_like(cnt)

    o[...] += pooled
    cnt[...] += c

    @pl.when(i == N // NBLK - 1)
    def _():
        o[...] = o[...] / jnp.maximum(cnt[...], 1.0)


def _out_proj_pool(x, batch2d, oW1, ob1, og, obeta, oW2, ob2):
    grid = (N // NBLK,)
    res = pl.pallas_call(
        _out_body,
        grid=grid,
        in_specs=[
            pl.BlockSpec((NBLK, H), lambda i: (i, 0)),
            pl.BlockSpec((NBLK, 1), lambda i: (i, 0)),
            pl.BlockSpec((H, H), lambda i: (0, 0)),
            pl.BlockSpec((1, H), lambda i: (0, 0)),
            pl.BlockSpec((1, H), lambda i: (0, 0)),
            pl.BlockSpec((1, H), lambda i: (0, 0)),
            pl.BlockSpec((H, OUT), lambda i: (0, 0)),
            pl.BlockSpec((1, OUT), lambda i: (0, 0)),
        ],
        out_specs=[
            pl.BlockSpec((B, OUT), lambda i: (0, 0)),
            pl.BlockSpec((B, OUT), lambda i: (0, 0)),
        ],
        out_shape=[
            jax.ShapeDtypeStruct((B, OUT), jnp.float32),
            jax.ShapeDtypeStruct((B, OUT), jnp.float32),
        ],
    )(x, batch2d, oW1, ob1.reshape(1, H), og.reshape(1, H),
      obeta.reshape(1, H), oW2, ob2.reshape(1, OUT))
    return res[0]


NC = 2
NS = 16
NW = NC * NS
EW = E // NW
CK = 80
GCK = 120
NCH = EW // CK
GNB = 3
GW = H // 2
NP = 10240
NPT = NP // NS
ZR = 16

_MESH = plsc.VectorSubcoreMesh(core_axis_name="c", subcore_axis_name="s",
                               num_cores=NC, num_subcores=NS)


def _make_gather(ecnt):
    ew = ecnt // NW
    nch = ew // GCK
    grounds = nch // GNB
    assert grounds * GNB == nch

    def body(p_hbm, q_hbm, dstr_hbm, srcr_hbm, gp_hbm, gq_hbm,
             idxd, idxs, *rest):
        bufp = list(rest[0:GNB])
        bufq = list(rest[GNB:2 * GNB])
        sems = rest[2 * GNB:]
        gsp, gsq, ssp, ssq = (sems[0:GNB], sems[GNB:2 * GNB],
                              sems[2 * GNB:3 * GNB], sems[3 * GNB:4 * GNB])
        c = lax.axis_index("c")
        s = lax.axis_index("s")
        w = s * NC + c
        ebase = w * ew
        pltpu.sync_copy(dstr_hbm.at[w], idxd)
        pltpu.sync_copy(srcr_hbm.at[w], idxs)

        def round_fn(t, carry):
            hp, hq = [], []
            for b in range(GNB):
                j = t * GNB + b

                @pl.when(t > 0)
                def _():
                    pltpu.make_async_copy(bufp[b], gp_hbm.at[pl.ds(ebase, GCK)],
                                          ssp[b]).wait()
                    pltpu.make_async_copy(bufq[b], gq_hbm.at[pl.ds(ebase, GCK)],
                                          ssq[b]).wait()
                hp.append(pltpu.async_copy(p_hbm.at[idxd.at[j]], bufp[b],
                                           gsp[b]))
                hq.append(pltpu.async_copy(q_hbm.at[idxs.at[j]], bufq[b],
                                           gsq[b]))
            for b in range(GNB):
                j = t * GNB + b
                start = ebase + j * GCK
                hp[b].wait()
                pltpu.async_copy(bufp[b], gp_hbm.at[pl.ds(start, GCK)], ssp[b])
                hq[b].wait()
                pltpu.async_copy(bufq[b], gq_hbm.at[pl.ds(start, GCK)], ssq[b])
            return carry

        lax.fori_loop(0, grounds, round_fn, 0)
        for b in range(GNB):
            pltpu.make_async_copy(bufp[b], gp_hbm.at[pl.ds(ebase, GCK)],
                                  ssp[b]).wait()
            pltpu.make_async_copy(bufq[b], gq_hbm.at[pl.ds(ebase, GCK)],
                                  ssq[b]).wait()

    return functools.partial(
        pl.kernel,
        out_type=[jax.ShapeDtypeStruct((ecnt, H), jnp.float32),
                  jax.ShapeDtypeStruct((ecnt, H), jnp.float32)],
        mesh=_MESH,
        scratch_types=([pltpu.VMEM((nch, GCK), jnp.int32)] * 2
                       + [pltpu.VMEM((GCK, H), jnp.float32)] * (2 * GNB)
                       + [pltpu.SemaphoreType.DMA] * (4 * GNB)),
    )(body)


def _zero_fill(ref, nrows):
    width = ref.shape[1]

    def row(i, carry):
        def col(k, carry2):
            ref[i, pl.ds(k * 16, 16)] = jnp.zeros((16,), jnp.float32)
            return carry2
        return lax.fori_loop(0, width // 16, col, carry)

    lax.fori_loop(0, nrows, row, 0)


def _zero_acc(acc, zbuf, s, zsem):
    _zero_fill(zbuf, ZR)

    def zcopy(k, carry):
        pltpu.async_copy(zbuf, acc.at[pl.ds(s * NPT + k * ZR, ZR)],
                         zsem).wait()
        return carry

    lax.fori_loop(0, NPT // ZR, zcopy, 0)


def _make_scatter(ecnt):
    ew = ecnt // NW
    nch = ew // CK
    SNB = 3

    def body(m_hbm, dstr_hbm, part_hbm, acc, idxd, zbuf, mb0, mb1, mb2,
             *sems):
        mbuf = [mb0, mb1, mb2]
        lsem = sems[0:SNB]
        zsem = sems[SNB]
        c = lax.axis_index("c")
        s = lax.axis_index("s")
        w = s * NC + c
        ebase = w * ew
        pltpu.sync_copy(dstr_hbm.at[w], idxd)
        _zero_acc(acc, zbuf, s, zsem)
        plsc.subcore_barrier()

        for b in range(min(SNB, nch)):
            pltpu.async_copy(m_hbm.at[pl.ds(ebase + b * CK, CK)], mbuf[b],
                             lsem[b])

        def chunk_fn(j, carry):
            for b in range(SNB):
                @pl.when(j % SNB == b)
                def _():
                    pltpu.make_async_copy(
                        m_hbm.at[pl.ds(ebase, CK)], mbuf[b], lsem[b]).wait()
                    pltpu.sync_copy(mbuf[b], acc.at[idxd.at[j]], add=True)

                    @pl.when(j + SNB < nch)
                    def _():
                        pltpu.async_copy(
                            m_hbm.at[pl.ds(ebase + (j + SNB) * CK, CK)],
                            mbuf[b], lsem[b])
            return carry

        lax.fori_loop(0, nch, chunk_fn, 0)
        plsc.subcore_barrier()

        def out_chunk(k, carry):
            pltpu.sync_copy(acc.at[pl.ds(s * NPT + k * CK, CK)], mbuf[0])
            pltpu.sync_copy(mbuf[0],
                            part_hbm.at[c, pl.ds(s * NPT + k * CK, CK)])
            return carry

        lax.fori_loop(0, NPT // CK, out_chunk, 0)

    return functools.partial(
        pl.kernel,
        out_type=jax.ShapeDtypeStruct((NC, NP, H), jnp.float32),
        mesh=_MESH,
        scratch_types=([pltpu.VMEM_SHARED((NP, H), jnp.float32),
                        pltpu.VMEM((nch, CK), jnp.int32),
                        pltpu.VMEM((ZR, H), jnp.float32)]
                       + [pltpu.VMEM((CK, H), jnp.float32)] * 3
                       + [pltpu.SemaphoreType.DMA] * 4),
    )(body)


EPAD = 322560
ESPLITS = ((0, 184320), (184320, 138240))
_GATHERS = {ec: _make_gather(ec) for _, ec in ESPLITS}
_SCATTERS = {ec: _make_scatter(ec) for _, ec in ESPLITS}


def _count_sc_body(dstr_hbm, part_hbm, acc, idxd, zbuf, onesb, obuf, zsem):
    c = lax.axis_index("c")
    s = lax.axis_index("s")
    w = s * NC + c
    pltpu.sync_copy(dstr_hbm.at[w], idxd)

    def ones_row(i, carry):
        onesb[i, pl.ds(0, H)] = jnp.ones((H,), jnp.float32)
        return carry

    lax.fori_loop(0, CK, ones_row, 0)
    _zero_acc(acc, zbuf, s, zsem)
    plsc.subcore_barrier()

    def chunk_fn(j, carry):
        pltpu.sync_copy(onesb, acc.at[idxd.at[j]], add=True)
        return carry

    lax.fori_loop(0, NCH, chunk_fn, 0)
    plsc.subcore_barrier()

    def out_chunk(k, carry):
        pltpu.sync_copy(acc.at[pl.ds(s * NPT + k * CK, CK)], obuf)
        pltpu.sync_copy(obuf, part_hbm.at[c, pl.ds(s * NPT + k * CK, CK)])
        return carry

    lax.fori_loop(0, NPT // CK, out_chunk, 0)


@functools.partial(
    pl.kernel,
    out_type=jax.ShapeDtypeStruct((NC, NP, H), jnp.float32),
    mesh=_MESH,
    scratch_types=[pltpu.VMEM_SHARED((NP, H), jnp.float32),
                   pltpu.VMEM((NCH, CK), jnp.int32),
                   pltpu.VMEM((ZR, H), jnp.float32),
                   pltpu.VMEM((CK, H), jnp.float32),
                   pltpu.VMEM((CK, H), jnp.float32),
                   pltpu.SemaphoreType.DMA],
)
def _count_sc(dstr, part, *scratch):
    _count_sc_body(dstr, part, *scratch)


def _gather_rows(P, Q, src, dst):
    return P[dst], Q[src]


def _scatter_partials(M, dst):
    s = jax.ops.segment_sum(M, dst, num_segments=NP)
    return jnp.stack([s, jnp.zeros_like(s)])


def _count_partials(dst):
    ones = jnp.ones((E, H), jnp.float32)
    c = jax.ops.segment_sum(ones, dst, num_segments=NP)
    return jnp.stack([c, jnp.zeros_like(c)])


def kernel(node_features, node_pos, edge_index, edge_attr, batch,
           W_in, b_in, ln_in_g, ln_in_b,
           edge_W1, edge_b1, edge_W2, edge_b2, edge_W3, edge_b3,
           node_W1, node_b1, node_W2, node_b2, ln_g, ln_b,
           out_W1, out_b1, out_ln_g, out_ln_b, out_W2, out_b2):
    src = edge_index[0]
    dst = edge_index[1]
    srcr = src.reshape(NW, NCH, CK)
    dstr = dst.reshape(NW, NCH, CK)
    npad = EPAD - E
    srcp = jnp.concatenate([src, jnp.zeros((npad,), jnp.int32)])
    dstp = jnp.concatenate([dst, jnp.full((npad,), N + 200, jnp.int32)])
    eap = jnp.concatenate([edge_attr, jnp.ones((npad, EDIM), jnp.float32)])
    sslices = [(e0, ec,
                srcp[e0:e0 + ec].reshape(NW, ec // NW // GCK, GCK),
                dstp[e0:e0 + ec].reshape(NW, ec // NW // GCK, GCK),
                dstp[e0:e0 + ec].reshape(NW, ec // NW // CK, CK),
                eap[e0:e0 + ec])
               for e0, ec in ESPLITS]
    batch2d = batch.reshape(N, 1)

    x = _input_proj(node_features, W_in, b_in, ln_in_g, ln_in_b)

    cp = _count_sc(dstr)
    c0, c1 = cp[0], cp[1]

    for l in range(L):
        W1a = edge_W1[l, 0:H]
        W1b = edge_W1[l, H:2 * H]
        W1c = edge_W1[l, 2 * H:]
        P, Q = _pq(x, W1a, W1b)
        parts = []
        for (e0, ec, srcr_g, dstr_g, dstr_s, ea_s) in sslices:
            gp, gq = _GATHERS[ec](P, Q, dstr_g, srcr_g)
            M = _edge_mlp(gp, gq, ea_s, W1c, edge_b1[l], edge_W2[l],
                          edge_b2[l], edge_W3[l], edge_b3[l])
            parts.append(_SCATTERS[ec](M, dstr_s))
        x = _node_update(x, parts[0][0], parts[0][1], parts[1][0],
                         parts[1][1], c0, c1,
                         node_W1[l, 0:H], node_W1[l, H:2 * H], node_b1[l],
                         node_W2[l], node_b2[l], ln_g[l], ln_b[l])

    return _out_proj_pool(x, batch2d, out_W1, out_b1, out_ln_g, out_ln_b,
                          out_W2, out_b2)


# speedup vs baseline: 1.3232x; 1.3232x over previous
import functools

import jax
import jax.numpy as jnp
from jax import lax
from jax.experimental import pallas as pl
from jax.experimental.pallas import tpu as pltpu
from jax.experimental.pallas import tpu_sc as plsc

N = 10000
E = 320000
D = 128
H = 128
EDIM = 4
L = 4
OUT = 128
B = 16

NBLK = 2000
EBLK = 3200


def _silu(x):
    return x * jax.nn.sigmoid(x)


def _ln(x, g, b):
    mu = jnp.mean(x, axis=-1, keepdims=True)
    var = jnp.mean((x - mu) ** 2, axis=-1, keepdims=True)
    return (x - mu) * jax.lax.rsqrt(var + 1e-5) * g + b


def _input_body(nf, w, b, g, beta, wa, wb, o, po, qo):
    h = jnp.dot(nf[...], w[...], preferred_element_type=jnp.float32) + b[...]
    xv = _silu(_ln(h, g[...], beta[...]))
    o[...] = xv
    po[...] = jnp.dot(xv, wa[...], preferred_element_type=jnp.float32)
    qo[...] = jnp.dot(xv, wb[...], preferred_element_type=jnp.float32)


def _input_proj(nf, W_in, b_in, g, beta, W1a, W1b):
    grid = (N // NBLK,)
    return pl.pallas_call(
        _input_body,
        grid=grid,
        in_specs=[
            pl.BlockSpec((NBLK, D), lambda i: (i, 0)),
            pl.BlockSpec((D, H), lambda i: (0, 0)),
            pl.BlockSpec((1, H), lambda i: (0, 0)),
            pl.BlockSpec((1, H), lambda i: (0, 0)),
            pl.BlockSpec((1, H), lambda i: (0, 0)),
            pl.BlockSpec((H, H), lambda i: (0, 0)),
            pl.BlockSpec((H, H), lambda i: (0, 0)),
        ],
        out_specs=[pl.BlockSpec((NBLK, H), lambda i: (i, 0))] * 3,
        out_shape=[jax.ShapeDtypeStruct((N, H), jnp.float32)] * 3,
    )(nf, W_in, b_in.reshape(1, H), g.reshape(1, H), beta.reshape(1, H),
      W1a, W1b)


def _pq_body(x, wa, wb, p, q):
    xv = x[...]
    p[...] = jnp.dot(xv, wa[...], preferred_element_type=jnp.float32)
    q[...] = jnp.dot(xv, wb[...], preferred_element_type=jnp.float32)


def _pq(x, W1a, W1b):
    grid = (N // NBLK,)
    return pl.pallas_call(
        _pq_body,
        grid=grid,
        in_specs=[
            pl.BlockSpec((NBLK, H), lambda i: (i, 0)),
            pl.BlockSpec((H, H), lambda i: (0, 0)),
            pl.BlockSpec((H, H), lambda i: (0, 0)),
        ],
        out_specs=[
            pl.BlockSpec((NBLK, H), lambda i: (i, 0)),
            pl.BlockSpec((NBLK, H), lambda i: (i, 0)),
        ],
        out_shape=[
            jax.ShapeDtypeStruct((N, H), jnp.float32),
            jax.ShapeDtypeStruct((N, H), jnp.float32),
        ],
    )(x, W1a, W1b)


def _edge_body(gp, gq, ea, w1c, b1, w2, b2, w3, b3, m_out):
    a = jnp.dot(ea[...], w1c[...], preferred_element_type=jnp.float32) + b1[...]
    m1 = _silu(gp[...] + gq[...] + a)
    m2 = _silu(jnp.dot(m1, w2[...], preferred_element_type=jnp.float32) + b2[...])
    m3 = jnp.dot(m2, w3[...], preferred_element_type=jnp.float32) + b3[...]
    dist = ea[:, 0:1]
    m_out[...] = m3 / (dist + 1e-8)


def _edge_mlp(gp, gq, ea, W1c, b1, W2, b2, W3, b3):
    ecnt = gp.shape[0]
    grid = (ecnt // EBLK,)
    return pl.pallas_call(
        _edge_body,
        grid=grid,
        in_specs=[
            pl.BlockSpec((EBLK, H), lambda i: (i, 0)),
            pl.BlockSpec((EBLK, H), lambda i: (i, 0)),
            pl.BlockSpec((EBLK, EDIM), lambda i: (i, 0)),
            pl.BlockSpec((EDIM, H), lambda i: (0, 0)),
            pl.BlockSpec((1, H), lambda i: (0, 0)),
            pl.BlockSpec((H, H), lambda i: (0, 0)),
            pl.BlockSpec((1, H), lambda i: (0, 0)),
            pl.BlockSpec((H, H), lambda i: (0, 0)),
            pl.BlockSpec((1, H), lambda i: (0, 0)),
        ],
        out_specs=pl.BlockSpec((EBLK, H), lambda i: (i, 0)),
        out_shape=jax.ShapeDtypeStruct((ecnt, H), jnp.float32),
    )(gp, gq, ea, W1c, b1.reshape(1, H), W2, b2.reshape(1, H), W3,
      b3.reshape(1, H))


def _node_body_pq(x, p0, p1, p2, p3, c0, c1, wa, wb, b1, w2, b2, g, beta,
                  wa2, wb2, xo, po, qo):
    xv = x[...]
    cnt = c0[:, 0:1] + c1[:, 0:1]
    deg = jnp.maximum(cnt, 1.0)
    agg = (p0[...] + p1[...] + p2[...] + p3[...]) / deg
    h = _silu(jnp.dot(xv, wa[...], preferred_element_type=jnp.float32)
              + jnp.dot(agg, wb[...], preferred_element_type=jnp.float32)
              + b1[...])
    h2 = jnp.dot(h, w2[...], preferred_element_type=jnp.float32) + b2[...]
    xn = _ln(h2 + xv, g[...], beta[...])
    xo[...] = xn
    po[...] = jnp.dot(xn, wa2[...], preferred_element_type=jnp.float32)
    qo[...] = jnp.dot(xn, wb2[...], preferred_element_type=jnp.float32)


def _node_update_pq(x, p0, p1, p2, p3, c0, c1, nW1a, nW1b, b1, W2, b2, g,
                    beta, W1a2, W1b2):
    grid = (N // NBLK,)
    return pl.pallas_call(
        _node_body_pq,
        grid=grid,
        in_specs=[
            pl.BlockSpec((NBLK, H), lambda i: (i, 0)),
            pl.BlockSpec((NBLK, H), lambda i: (i, 0)),
            pl.BlockSpec((NBLK, H), lambda i: (i, 0)),
            pl.BlockSpec((NBLK, H), lambda i: (i, 0)),
            pl.BlockSpec((NBLK, H), lambda i: (i, 0)),
            pl.BlockSpec((NBLK, H), lambda i: (i, 0)),
            pl.BlockSpec((NBLK, H), lambda i: (i, 0)),
            pl.BlockSpec((H, H), lambda i: (0, 0)),
            pl.BlockSpec((H, H), lambda i: (0, 0)),
            pl.BlockSpec((1, H), lambda i: (0, 0)),
            pl.BlockSpec((H, H), lambda i: (0, 0)),
            pl.BlockSpec((1, H), lambda i: (0, 0)),
            pl.BlockSpec((1, H), lambda i: (0, 0)),
            pl.BlockSpec((1, H), lambda i: (0, 0)),
            pl.BlockSpec((H, H), lambda i: (0, 0)),
            pl.BlockSpec((H, H), lambda i: (0, 0)),
        ],
        out_specs=[pl.BlockSpec((NBLK, H), lambda i: (i, 0))] * 3,
        out_shape=[jax.ShapeDtypeStruct((N, H), jnp.float32)] * 3,
    )(x, p0, p1, p2, p3, c0, c1, nW1a, nW1b, b1.reshape(1, H), W2,
      b2.reshape(1, H), g.reshape(1, H), beta.reshape(1, H), W1a2, W1b2)


def _node_body(x, p0, p1, p2, p3, c0, c1, wa, wb, b1, w2, b2, g, beta, xo):
    xv = x[...]
    cnt = c0[:, 0:1] + c1[:, 0:1]
    deg = jnp.maximum(cnt, 1.0)
    agg = (p0[...] + p1[...] + p2[...] + p3[...]) / deg
    h = _silu(jnp.dot(xv, wa[...], preferred_element_type=jnp.float32)
              + jnp.dot(agg, wb[...], preferred_element_type=jnp.float32)
              + b1[...])
    h2 = jnp.dot(h, w2[...], preferred_element_type=jnp.float32) + b2[...]
    xo[...] = _ln(h2 + xv, g[...], beta[...])


def _node_update(x, p0, p1, p2, p3, c0, c1, nW1a, nW1b, b1, W2, b2, g, beta):
    grid = (N // NBLK,)
    return pl.pallas_call(
        _node_body,
        grid=grid,
        in_specs=[
            pl.BlockSpec((NBLK, H), lambda i: (i, 0)),
            pl.BlockSpec((NBLK, H), lambda i: (i, 0)),
            pl.BlockSpec((NBLK, H), lambda i: (i, 0)),
            pl.BlockSpec((NBLK, H), lambda i: (i, 0)),
            pl.BlockSpec((NBLK, H), lambda i: (i, 0)),
            pl.BlockSpec((NBLK, H), lambda i: (i, 0)),
            pl.BlockSpec((NBLK, H), lambda i: (i, 0)),
            pl.BlockSpec((H, H), lambda i: (0, 0)),
            pl.BlockSpec((H, H), lambda i: (0, 0)),
            pl.BlockSpec((1, H), lambda i: (0, 0)),
            pl.BlockSpec((H, H), lambda i: (0, 0)),
            pl.BlockSpec((1, H), lambda i: (0, 0)),
            pl.BlockSpec((1, H), lambda i: (0, 0)),
            pl.BlockSpec((1, H), lambda i: (0, 0)),
        ],
        out_specs=pl.BlockSpec((NBLK, H), lambda i: (i, 0)),
        out_shape=jax.ShapeDtypeStruct((N, H), jnp.float32),
    )(x, p0, p1, p2, p3, c0, c1, nW1a, nW1b, b1.reshape(1, H), W2,
      b2.reshape(1, H), g.reshape(1, H), beta.reshape(1, H))


def _out_body(x, batch, w1, b1, g, beta, w2, b2, o, cnt):
    i = pl.program_id(0)
    y = _silu(_ln(jnp.dot(x[...], w1[...], preferred_element_type=jnp.float32)
                  + b1[...], g[...], beta[...]))
    z = jnp.dot(y, w2[...], preferred_element_type=jnp.float32) + b2[...]
    bb = batch[...]
    iota = jax.lax.broadcasted_iota(jnp.int32, (NBLK, B), 1)
    oh = (bb == iota).astype(jnp.float32)
    pooled = jax.lax.dot_general(oh, z, (((0,), (0,)), ((), ())),
                                 preferred_element_type=jnp.float32)
    ones = jnp.ones((NBLK, OUT), jnp.float32)
    c = jax.lax.dot_general(oh, ones, (((0,), (0,)), ((), ())),
                            preferred_element_type=jnp.float32)

    @pl.when(i == 0)
    def _():
        o[...] = jnp.zeros_like(o)
        cnt[...] = jnp.zeros_like(cnt)

    o[...] += pooled
    cnt[...] += c

    @pl.when(i == N // NBLK - 1)
    def _():
        o[...] = o[...] / jnp.maximum(cnt[...], 1.0)


def _out_proj_pool(x, batch2d, oW1, ob1, og, obeta, oW2, ob2):
    grid = (N // NBLK,)
    res = pl.pallas_call(
        _out_body,
        grid=grid,
        in_specs=[
            pl.BlockSpec((NBLK, H), lambda i: (i, 0)),
            pl.BlockSpec((NBLK, 1), lambda i: (i, 0)),
            pl.BlockSpec((H, H), lambda i: (0, 0)),
            pl.BlockSpec((1, H), lambda i: (0, 0)),
            pl.BlockSpec((1, H), lambda i: (0, 0)),
            pl.BlockSpec((1, H), lambda i: (0, 0)),
            pl.BlockSpec((H, OUT), lambda i: (0, 0)),
            pl.BlockSpec((1, OUT), lambda i: (0, 0)),
        ],
        out_specs=[
            pl.BlockSpec((B, OUT), lambda i: (0, 0)),
            pl.BlockSpec((B, OUT), lambda i: (0, 0)),
        ],
        out_shape=[
            jax.ShapeDtypeStruct((B, OUT), jnp.float32),
            jax.ShapeDtypeStruct((B, OUT), jnp.float32),
        ],
    )(x, batch2d, oW1, ob1.reshape(1, H), og.reshape(1, H),
      obeta.reshape(1, H), oW2, ob2.reshape(1, OUT))
    return res[0]


NC = 2
NS = 16
NW = NC * NS
EW = E // NW
CK = 80
NCH = EW // CK
GNB = 5
GROUNDS = NCH // GNB
GW = H // 2
NP = 10240
NPT = NP // NS
ZR = 16

_MESH = plsc.VectorSubcoreMesh(core_axis_name="c", subcore_axis_name="s",
                               num_cores=NC, num_subcores=NS)


def _make_gather(ecnt):
    ew = ecnt // NW
    nch = ew // CK
    grounds = nch // GNB
    assert grounds * GNB == nch

    def body(p_hbm, q_hbm, dstr_hbm, srcr_hbm, gp_hbm, gq_hbm,
             idxd, idxs, *rest):
        bufp = list(rest[0:GNB])
        bufq = list(rest[GNB:2 * GNB])
        sems = rest[2 * GNB:]
        gsp, gsq, ssp, ssq = (sems[0:GNB], sems[GNB:2 * GNB],
                              sems[2 * GNB:3 * GNB], sems[3 * GNB:4 * GNB])
        c = lax.axis_index("c")
        s = lax.axis_index("s")
        w = s * NC + c
        ebase = w * ew
        pltpu.sync_copy(dstr_hbm.at[w], idxd)
        pltpu.sync_copy(srcr_hbm.at[w], idxs)

        def round_fn(t, carry):
            hp, hq = [], []
            for b in range(GNB):
                j = t * GNB + b

                @pl.when(t > 0)
                def _():
                    pltpu.make_async_copy(bufp[b], gp_hbm.at[pl.ds(ebase, CK)],
                                          ssp[b]).wait()
                    pltpu.make_async_copy(bufq[b], gq_hbm.at[pl.ds(ebase, CK)],
                                          ssq[b]).wait()
                hp.append(pltpu.async_copy(p_hbm.at[idxd.at[j]], bufp[b],
                                           gsp[b]))
                hq.append(pltpu.async_copy(q_hbm.at[idxs.at[j]], bufq[b],
                                           gsq[b]))
            for b in range(GNB):
                j = t * GNB + b
                start = ebase + j * CK
                hp[b].wait()
                pltpu.async_copy(bufp[b], gp_hbm.at[pl.ds(start, CK)], ssp[b])
                hq[b].wait()
                pltpu.async_copy(bufq[b], gq_hbm.at[pl.ds(start, CK)], ssq[b])
            return carry

        lax.fori_loop(0, grounds, round_fn, 0)
        for b in range(GNB):
            pltpu.make_async_copy(bufp[b], gp_hbm.at[pl.ds(ebase, CK)],
                                  ssp[b]).wait()
            pltpu.make_async_copy(bufq[b], gq_hbm.at[pl.ds(ebase, CK)],
                                  ssq[b]).wait()

    return functools.partial(
        pl.kernel,
        out_type=[jax.ShapeDtypeStruct((ecnt, H), jnp.float32),
                  jax.ShapeDtypeStruct((ecnt, H), jnp.float32)],
        mesh=_MESH,
        scratch_types=([pltpu.VMEM((nch, CK), jnp.int32)] * 2
                       + [pltpu.VMEM((CK, H), jnp.float32)] * (2 * GNB)
                       + [pltpu.SemaphoreType.DMA] * (4 * GNB)),
    )(body)


def _zero_fill(ref, nrows):
    width = ref.shape[1]

    def row(i, carry):
        def col(k, carry2):
            ref[i, pl.ds(k * 16, 16)] = jnp.zeros((16,), jnp.float32)
            return carry2
        return lax.fori_loop(0, width // 16, col, carry)

    lax.fori_loop(0, nrows, row, 0)


def _zero_acc(acc, zbuf, s, zsem):
    _zero_fill(zbuf, ZR)

    def zcopy(k, carry):
        pltpu.async_copy(zbuf, acc.at[pl.ds(s * NPT + k * ZR, ZR)],
                         zsem).wait()
        return carry

    lax.fori_loop(0, NPT // ZR, zcopy, 0)


def _make_scatter(ecnt):
    ew = ecnt // NW
    nch = ew // CK
    SNB = 3

    def body(m_hbm, dstr_hbm, part_hbm, acc, idxd, zbuf, mb0, mb1, mb2,
             *sems):
        mbuf = [mb0, mb1, mb2]
        lsem = sems[0:SNB]
        zsem = sems[SNB]
        c = lax.axis_index("c")
        s = lax.axis_index("s")
        w = s * NC + c
        ebase = w * ew
        pltpu.sync_copy(dstr_hbm.at[w], idxd)
        _zero_acc(acc, zbuf, s, zsem)
        plsc.subcore_barrier()

        for b in range(min(SNB, nch)):
            pltpu.async_copy(m_hbm.at[pl.ds(ebase + b * CK, CK)], mbuf[b],
                             lsem[b])

        def chunk_fn(j, carry):
            for b in range(SNB):
                @pl.when(j % SNB == b)
                def _():
                    pltpu.make_async_copy(
                        m_hbm.at[pl.ds(ebase, CK)], mbuf[b], lsem[b]).wait()
                    pltpu.sync_copy(mbuf[b], acc.at[idxd.at[j]], add=True)

                    @pl.when(j + SNB < nch)
                    def _():
                        pltpu.async_copy(
                            m_hbm.at[pl.ds(ebase + (j + SNB) * CK, CK)],
                            mbuf[b], lsem[b])
            return carry

        lax.fori_loop(0, nch, chunk_fn, 0)
        plsc.subcore_barrier()

        def out_chunk(k, carry):
            pltpu.sync_copy(acc.at[pl.ds(s * NPT + k * CK, CK)], mbuf[0])
            pltpu.sync_copy(mbuf[0],
                            part_hbm.at[c, pl.ds(s * NPT + k * CK, CK)])
            return carry

        lax.fori_loop(0, NPT // CK, out_chunk, 0)

    return functools.partial(
        pl.kernel,
        out_type=jax.ShapeDtypeStruct((NC, NP, H), jnp.float32),
        mesh=_MESH,
        scratch_types=([pltpu.VMEM_SHARED((NP, H), jnp.float32),
                        pltpu.VMEM((nch, CK), jnp.int32),
                        pltpu.VMEM((ZR, H), jnp.float32)]
                       + [pltpu.VMEM((CK, H), jnp.float32)] * 3
                       + [pltpu.SemaphoreType.DMA] * 4),
    )(body)


ESPLITS = ((0, 192000), (192000, 128000))
_GATHERS = {ec: _make_gather(ec) for _, ec in ESPLITS}
_SCATTERS = {ec: _make_scatter(ec) for _, ec in ESPLITS}


def _count_sc_body(dstr_hbm, part_hbm, acc, idxd, zbuf, onesb, obuf, zsem):
    c = lax.axis_index("c")
    s = lax.axis_index("s")
    w = s * NC + c
    pltpu.sync_copy(dstr_hbm.at[w], idxd)

    def ones_row(i, carry):
        onesb[i, pl.ds(0, H)] = jnp.ones((H,), jnp.float32)
        return carry

    lax.fori_loop(0, CK, ones_row, 0)
    _zero_acc(acc, zbuf, s, zsem)
    plsc.subcore_barrier()

    def chunk_fn(j, carry):
        pltpu.sync_copy(onesb, acc.at[idxd.at[j]], add=True)
        return carry

    lax.fori_loop(0, NCH, chunk_fn, 0)
    plsc.subcore_barrier()

    def out_chunk(k, carry):
        pltpu.sync_copy(acc.at[pl.ds(s * NPT + k * CK, CK)], obuf)
        pltpu.sync_copy(obuf, part_hbm.at[c, pl.ds(s * NPT + k * CK, CK)])
        return carry

    lax.fori_loop(0, NPT // CK, out_chunk, 0)


@functools.partial(
    pl.kernel,
    out_type=jax.ShapeDtypeStruct((NC, NP, H), jnp.float32),
    mesh=_MESH,
    scratch_types=[pltpu.VMEM_SHARED((NP, H), jnp.float32),
                   pltpu.VMEM((NCH, CK), jnp.int32),
                   pltpu.VMEM((ZR, H), jnp.float32),
                   pltpu.VMEM((CK, H), jnp.float32),
                   pltpu.VMEM((CK, H), jnp.float32),
                   pltpu.SemaphoreType.DMA],
)
def _count_sc(dstr, part, *scratch):
    _count_sc_body(dstr, part, *scratch)


def _gather_rows(P, Q, src, dst):
    return P[dst], Q[src]


def _scatter_partials(M, dst):
    s = jax.ops.segment_sum(M, dst, num_segments=NP)
    return jnp.stack([s, jnp.zeros_like(s)])


def _count_partials(dst):
    ones = jnp.ones((E, H), jnp.float32)
    c = jax.ops.segment_sum(ones, dst, num_segments=NP)
    return jnp.stack([c, jnp.zeros_like(c)])


def kernel(node_features, node_pos, edge_index, edge_attr, batch,
           W_in, b_in, ln_in_g, ln_in_b,
           edge_W1, edge_b1, edge_W2, edge_b2, edge_W3, edge_b3,
           node_W1, node_b1, node_W2, node_b2, ln_g, ln_b,
           out_W1, out_b1, out_ln_g, out_ln_b, out_W2, out_b2):
    src = edge_index[0]
    dst = edge_index[1]
    srcr = src.reshape(NW, NCH, CK)
    dstr = dst.reshape(NW, NCH, CK)
    sslices = [(e0, ec,
                src[e0:e0 + ec].reshape(NW, ec // NW // CK, CK),
                dst[e0:e0 + ec].reshape(NW, ec // NW // CK, CK),
                edge_attr[e0:e0 + ec])
               for e0, ec in ESPLITS]
    batch2d = batch.reshape(N, 1)

    x, P, Q = _input_proj(node_features, W_in, b_in, ln_in_g, ln_in_b,
                          edge_W1[0, 0:H], edge_W1[0, H:2 * H])

    cp = _count_sc(dstr)
    c0, c1 = cp[0], cp[1]

    for l in range(L):
        W1c = edge_W1[l, 2 * H:]
        parts = []
        for (e0, ec, srcr_s, dstr_s, ea_s) in sslices:
            gp, gq = _GATHERS[ec](P, Q, dstr_s, srcr_s)
            M = _edge_mlp(gp, gq, ea_s, W1c, edge_b1[l], edge_W2[l],
                          edge_b2[l], edge_W3[l], edge_b3[l])
            parts.append(_SCATTERS[ec](M, dstr_s))
        if l < L - 1:
            x, P, Q = _node_update_pq(
                x, parts[0][0], parts[0][1], parts[1][0], parts[1][1], c0, c1,
                node_W1[l, 0:H], node_W1[l, H:2 * H], node_b1[l],
                node_W2[l], node_b2[l], ln_g[l], ln_b[l],
                edge_W1[l + 1, 0:H], edge_W1[l + 1, H:2 * H])
        else:
            x = _node_update(x, parts[0][0], parts[0][1], parts[1][0],
                             parts[1][1], c0, c1,
                             node_W1[l, 0:H], node_W1[l, H:2 * H],
                             node_b1[l], node_W2[l], node_b2[l], ln_g[l],
                             ln_b[l])

    return _out_proj_pool(x, batch2d, out_W1, out_b1, out_ln_g, out_ln_b,
                          out_W2, out_b2)
